# Initial kernel scaffold; baseline (speedup 1.0000x reference)
import jax
import jax.numpy as jnp
from jax.experimental import pallas as pl


def kernel(token_ids, embedding):
    raise NotImplementedError("write your pallas kernel here")


# speedup vs baseline: 1.6836x; 1.6836x over previous
import functools

import jax
import jax.numpy as jnp
from jax import lax
from jax.experimental import pallas as pl
from jax.experimental.pallas import tpu as pltpu
from jax.experimental.pallas import tpu_sc as plsc

_NUM_CORES = 2
_NUM_SUBCORES = 16
_NW = _NUM_CORES * _NUM_SUBCORES
_CHUNK = 128


@functools.lru_cache(maxsize=None)
def _make_gather(V, D, B):
    per_w = B // _NW
    n_chunks = per_w // _CHUNK
    mesh = plsc.VectorSubcoreMesh(core_axis_name="c", subcore_axis_name="s")

    @functools.partial(
        pl.kernel,
        out_type=jax.ShapeDtypeStruct((B, D), jnp.float32),
        mesh=mesh,
        scratch_types=[
            pltpu.VMEM((n_chunks, _CHUNK), jnp.int32),
            pltpu.VMEM((_CHUNK, D), jnp.float32),
            pltpu.SemaphoreType.DMA,
        ],
        compiler_params=pltpu.CompilerParams(use_tc_tiling_on_sc=False),
    )
    def k(table_hbm, idx_hbm, out_hbm, idx_v, rows_v, sem):
        wid = lax.axis_index("s") * _NUM_CORES + lax.axis_index("c")
        base = wid * per_w
        pltpu.sync_copy(idx_hbm.at[wid], idx_v)

        def body(j, carry):
            pltpu.async_copy(table_hbm.at[idx_v.at[j]], rows_v, sem).wait()
            pltpu.sync_copy(rows_v, out_hbm.at[pl.ds(base + j * _CHUNK, _CHUNK)])
            return carry

        lax.fori_loop(0, n_chunks, body, 0)

    return k


def kernel(token_ids, embedding):
    Bt, H = token_ids.shape
    V, D = embedding.shape
    B = Bt * H
    idx = token_ids.reshape(_NW, B // _NW // _CHUNK, _CHUNK).astype(jnp.int32)
    out = _make_gather(V, D, B)(embedding, idx)
    return out.reshape(Bt, H, D)


# speedup vs baseline: 1.8763x; 1.1144x over previous
import functools

import jax
import jax.numpy as jnp
from jax import lax
from jax.experimental import pallas as pl
from jax.experimental.pallas import tpu as pltpu
from jax.experimental.pallas import tpu_sc as plsc

_NUM_CORES = 2
_NUM_SUBCORES = 16
_NW = _NUM_CORES * _NUM_SUBCORES
_CHUNK = 128
_NBUF = 8
_GD = 4


@functools.lru_cache(maxsize=None)
def _make_gather(V, D, B):
    per_w = B // _NW
    n_chunks = per_w // _CHUNK
    assert n_chunks % _NBUF == 0 and n_chunks >= 2 * _NBUF
    mesh = plsc.VectorSubcoreMesh(core_axis_name="c", subcore_axis_name="s")

    @functools.partial(
        pl.kernel,
        out_type=jax.ShapeDtypeStruct((B, D), jnp.float32),
        mesh=mesh,
        scratch_types=[
            pltpu.VMEM((n_chunks, _CHUNK), jnp.int32),
            pltpu.VMEM((_NBUF, _CHUNK, D), jnp.float32),
            pltpu.SemaphoreType.DMA,
            pltpu.SemaphoreType.DMA,
        ],
        compiler_params=pltpu.CompilerParams(use_tc_tiling_on_sc=False),
    )
    def k(table_hbm, idx_hbm, out_hbm, idx_v, rows_v, gsem, wsem):
        wid = lax.axis_index("s") * _NUM_CORES + lax.axis_index("c")
        base = wid * per_w
        pltpu.sync_copy(idx_hbm.at[wid], idx_v)

        def g_start(m, s):
            pltpu.async_copy(table_hbm.at[idx_v.at[m]], rows_v.at[s], gsem)

        def g_wait(m, s):
            pltpu.make_async_copy(table_hbm.at[idx_v.at[m]], rows_v.at[s], gsem).wait()

        def w_start(m, s):
            pltpu.async_copy(
                rows_v.at[s], out_hbm.at[pl.ds(base + m * _CHUNK, _CHUNK)], wsem)

        def w_wait(m, s):
            pltpu.make_async_copy(
                rows_v.at[s], out_hbm.at[pl.ds(base + m * _CHUNK, _CHUNK)], wsem).wait()

        for m in range(_GD):
            g_start(m, m)
        for m in range(_NBUF - _GD):
            g_wait(m, m % _NBUF)
            w_start(m, m % _NBUF)
            g_start(m + _GD, (m + _GD) % _NBUF)

        n_main = (n_chunks - _NBUF) // _NBUF

        def body(g, carry):
            m0 = (_NBUF - _GD) + g * _NBUF
            for b in range(_NBUF):
                m = m0 + b
                s = (_NBUF - _GD + b) % _NBUF
                sn = b
                g_wait(m, s)
                w_start(m, s)
                w_wait(m + _GD - _NBUF, sn)
                g_start(m + _GD, sn)
            return carry

        lax.fori_loop(0, n_main, body, 0)

        for i in range(_GD):
            m = n_chunks - _GD + i
            g_wait(m, m % _NBUF)
            w_start(m, m % _NBUF)
        for i in range(_NBUF):
            m = n_chunks - _NBUF + i
            w_wait(m, m % _NBUF)

    return k


def kernel(token_ids, embedding):
    Bt, H = token_ids.shape
    V, D = embedding.shape
    B = Bt * H
    idx = token_ids.reshape(_NW, B // _NW // _CHUNK, _CHUNK).astype(jnp.int32)
    out = _make_gather(V, D, B)(embedding, idx)
    return out.reshape(Bt, H, D)


# speedup vs baseline: 2.4976x; 1.3311x over previous
import functools

import jax
import jax.numpy as jnp
from jax import lax
from jax.experimental import pallas as pl
from jax.experimental.pallas import tpu as pltpu
from jax.experimental.pallas import tpu_sc as plsc

_NUM_CORES = 2
_NUM_SUBCORES = 16
_NW = _NUM_CORES * _NUM_SUBCORES
_NBUF = 8
_GD = 4


@functools.lru_cache(maxsize=None)
def _make_gather(V, D, Bt, H, Hp, Dp):
    rows_per_w = Bt // _NW
    mesh = plsc.VectorSubcoreMesh(core_axis_name="c", subcore_axis_name="s")

    @functools.partial(
        pl.kernel,
        out_type=jax.ShapeDtypeStruct((Bt, Hp, Dp), jnp.float32),
        mesh=mesh,
        scratch_types=[
            pltpu.VMEM((rows_per_w, H), jnp.int32),
            pltpu.VMEM((_NBUF, H, D), jnp.float32),
            pltpu.SemaphoreType.DMA,
            pltpu.SemaphoreType.DMA,
        ],
        compiler_params=pltpu.CompilerParams(use_tc_tiling_on_sc=False),
    )
    def k(table_hbm, idx_hbm, out_hbm, idx_v, rows_v, gsem, wsem):
        wid = lax.axis_index("s") * _NUM_CORES + lax.axis_index("c")
        rbase = wid * rows_per_w
        pltpu.sync_copy(idx_hbm.at[wid], idx_v)

        def g_start(m, s):
            pltpu.async_copy(table_hbm.at[idx_v.at[m]], rows_v.at[s], gsem)

        def g_wait(m, s):
            pltpu.make_async_copy(table_hbm.at[idx_v.at[m]], rows_v.at[s], gsem).wait()

        def w_start(m, s):
            pltpu.async_copy(
                rows_v.at[s],
                out_hbm.at[rbase + m, pl.ds(0, H), pl.ds(0, D)], wsem)

        def w_wait(m, s):
            pltpu.make_async_copy(
                rows_v.at[s],
                out_hbm.at[rbase + m, pl.ds(0, H), pl.ds(0, D)], wsem).wait()

        for m in range(_GD):
            g_start(m, m)
        for m in range(_NBUF - _GD):
            g_wait(m, m % _NBUF)
            w_start(m, m % _NBUF)
            g_start(m + _GD, (m + _GD) % _NBUF)

        n_main = (rows_per_w - _NBUF) // _NBUF

        def body(g, carry):
            m0 = (_NBUF - _GD) + g * _NBUF
            for b in range(_NBUF):
                m = m0 + b
                s = (_NBUF - _GD + b) % _NBUF
                sn = b
                g_wait(m, s)
                w_start(m, s)
                w_wait(m + _GD - _NBUF, sn)
                g_start(m + _GD, sn)
            return carry

        lax.fori_loop(0, n_main, body, 0)

        for i in range(_GD):
            m = rows_per_w - _GD + i
            g_wait(m, m % _NBUF)
            w_start(m, m % _NBUF)
        for i in range(_NBUF):
            m = rows_per_w - _NBUF + i
            w_wait(m, m % _NBUF)

    return k


def kernel(token_ids, embedding):
    Bt, H = token_ids.shape
    V, D = embedding.shape
    Hp = (H + 7) // 8 * 8
    Dp = 128
    idx = token_ids.reshape(_NW, Bt // _NW, H).astype(jnp.int32)
    out = _make_gather(V, D, Bt, H, Hp, Dp)(embedding, idx)
    return out[:, :H, :D]
